# Initial kernel scaffold; baseline (speedup 1.0000x reference)
import jax
import jax.numpy as jnp
from jax.experimental import pallas as pl


def kernel(x1, edge_index1, x2, edge_index2, W_l, b_l, W_r):
    raise NotImplementedError("write your pallas kernel here")


# speedup vs baseline: 6.7654x; 6.7654x over previous
import functools

import jax
import jax.numpy as jnp
from jax import lax
from jax.experimental import pallas as pl
from jax.experimental.pallas import tpu as pltpu
from jax.experimental.pallas import tpu_sc as plsc

NC = 2
NS = 16
LANES = 16
CHUNK = 80
DP = 144


def _tc_prep(x1, x2, N, D):
    BLK = 1000

    def body(x1_ref, x2_ref, o1_ref, o2_ref):
        pad = jnp.where(lax.broadcasted_iota(jnp.int32, (BLK, DP - D), 1) == 0,
                        1.0, 0.0)
        o1_ref[...] = jnp.concatenate([x1_ref[...], pad], axis=1)
        o2_ref[...] = jnp.concatenate([x2_ref[...], pad], axis=1)

    return pl.pallas_call(
        body,
        grid=(N // BLK,),
        in_specs=[
            pl.BlockSpec((BLK, D), lambda i: (i, 0)),
            pl.BlockSpec((BLK, D), lambda i: (i, 0)),
        ],
        out_specs=[
            pl.BlockSpec((BLK, DP), lambda i: (i, 0)),
            pl.BlockSpec((BLK, DP), lambda i: (i, 0)),
        ],
        out_shape=[
            jax.ShapeDtypeStruct((N, DP), jnp.float32),
            jax.ShapeDtypeStruct((N, DP), jnp.float32),
        ],
    )(x1, x2)


def _sc_aggregate(ytab1, ytab2, ei1, ei2, N, E):
    edges_per_tile = E // NS
    n_chunks = edges_per_tile // CHUNK
    IBLK = 50
    n_iblk = n_chunks // IBLK
    rows_per_tile = N // NS

    mesh = plsc.VectorSubcoreMesh(core_axis_name="c", subcore_axis_name="s")

    @functools.partial(
        pl.kernel,
        out_type=jax.ShapeDtypeStruct((2 * N, DP), jnp.float32),
        mesh=mesh,
        scratch_types=[
            pltpu.VMEM((IBLK, CHUNK), jnp.int32),
            pltpu.VMEM((IBLK, CHUNK), jnp.int32),
            pltpu.VMEM((CHUNK, DP), jnp.float32),
            pltpu.VMEM_SHARED((N, DP), jnp.float32),
            pltpu.SemaphoreType.DMA,
            pltpu.SemaphoreType.DMA,
        ],
        compiler_params=pltpu.CompilerParams(use_tc_tiling_on_sc=False),
    )
    def k(ytab1_hbm, ytab2_hbm, ei1_hbm, ei2_hbm, out_hbm,
          src_v, dst_v, rows_v, agg_sh, gsem, ssem):
        c = lax.axis_index("c")
        s = lax.axis_index("s")

        z16 = jnp.zeros((LANES,), jnp.float32)
        for r in range(CHUNK):
            for q in range(DP // LANES):
                rows_v[r, pl.ds(q * LANES, LANES)] = z16
        base = s * rows_per_tile
        for i in range(rows_per_tile // CHUNK):
            pltpu.sync_copy(rows_v, agg_sh.at[pl.ds(base + i * CHUNK, CHUNK)])
        rem = rows_per_tile % CHUNK
        if rem:
            pltpu.sync_copy(
                rows_v.at[pl.ds(0, rem)],
                agg_sh.at[pl.ds(base + rows_per_tile - rem, rem)])
        plsc.subcore_barrier()

        def make_outer(tab, ei):
            def outer(b, _):
                pltpu.sync_copy(ei.at[0, s].at[pl.ds(b * IBLK, IBLK)], src_v)
                pltpu.sync_copy(ei.at[1, s].at[pl.ds(b * IBLK, IBLK)], dst_v)

                def body(j, _):
                    pltpu.async_copy(tab.at[src_v.at[j]], rows_v, gsem).wait()
                    pltpu.async_copy(rows_v, agg_sh.at[dst_v.at[j]], ssem,
                                     add=True).wait()
                    return ()
                lax.fori_loop(0, IBLK, body, ())
                return ()
            return outer

        @pl.when(c == 0)
        def _():
            lax.fori_loop(0, n_iblk, make_outer(ytab1_hbm, ei1_hbm), ())

        @pl.when(c == 1)
        def _():
            lax.fori_loop(0, n_iblk, make_outer(ytab2_hbm, ei2_hbm), ())

        plsc.subcore_barrier()

        pltpu.sync_copy(
            agg_sh.at[pl.ds(s * rows_per_tile, rows_per_tile)],
            out_hbm.at[pl.ds(c * N + s * rows_per_tile, rows_per_tile)],
        )

    return k(ytab1, ytab2, ei1, ei2)


def _tc_finish(agg, x1, x2, W_l, b_l, W_r, N):
    BLK = 1000
    D = x1.shape[1]
    nblk = N // BLK

    def body(agg_ref, x1_ref, x2_ref, wl_ref, bl_ref, wr_ref, o_ref):
        g = pl.program_id(0)
        x = jnp.where(g == 0, x1_ref[...], x2_ref[...])
        a = agg_ref[...]
        cnt = jnp.maximum(a[:, D:D + 1], 1.0)
        mean = a[:, :D] / cnt
        dn = (((1,), (1,)), ((), ()))
        o_ref[...] = (
            lax.dot_general(mean, wl_ref[...], dn, preferred_element_type=jnp.float32)
            + lax.dot_general(x, wr_ref[...], dn, preferred_element_type=jnp.float32)
            + bl_ref[...]
        )

    return pl.pallas_call(
        body,
        grid=(2, nblk),
        in_specs=[
            pl.BlockSpec((BLK, DP), lambda g, i: (g * nblk + i, 0)),
            pl.BlockSpec((BLK, D), lambda g, i: (i, 0)),
            pl.BlockSpec((BLK, D), lambda g, i: (i, 0)),
            pl.BlockSpec((D, D), lambda g, i: (0, 0)),
            pl.BlockSpec((1, D), lambda g, i: (0, 0)),
            pl.BlockSpec((D, D), lambda g, i: (0, 0)),
        ],
        out_specs=pl.BlockSpec((BLK, D), lambda g, i: (g * nblk + i, 0)),
        out_shape=jax.ShapeDtypeStruct((2 * N, D), jnp.float32),
    )(agg, x1, x2, W_l, b_l.reshape(1, D), W_r)


def kernel(x1, edge_index1, x2, edge_index2, W_l, b_l, W_r):
    N, D = x1.shape
    E = edge_index1.shape[1]
    n_chunks = E // NS // CHUNK

    ytab1, ytab2 = _tc_prep(x1, x2, N, D)
    ei1 = edge_index1.reshape(2, NS, n_chunks, CHUNK)
    ei2 = edge_index2.reshape(2, NS, n_chunks, CHUNK)
    agg = _sc_aggregate(ytab1, ytab2, ei1, ei2, N, E)
    out = _tc_finish(agg, x1, x2, W_l, b_l, W_r, N)
    return out[:N], out[N:]


# speedup vs baseline: 14.2214x; 2.1021x over previous
import functools

import jax
import jax.numpy as jnp
from jax import lax
from jax.experimental import pallas as pl
from jax.experimental.pallas import tpu as pltpu
from jax.experimental.pallas import tpu_sc as plsc

NC = 2
NS = 16
LANES = 16
CHUNK = 80
CW = 16


def _sc_aggregate(x1, x2, ei1, ei2, N, D, E):
    edges_per_tile = E // NS
    n_chunks = edges_per_tile // CHUNK
    IBLK = 25
    n_iblk = n_chunks // IBLK
    rows_per_tile = N // NS
    NBUF = 3

    mesh = plsc.VectorSubcoreMesh(core_axis_name="c", subcore_axis_name="s")

    @functools.partial(
        pl.kernel,
        out_type=(
            jax.ShapeDtypeStruct((2 * N, D), jnp.float32),
            jax.ShapeDtypeStruct((2 * N, CW), jnp.float32),
        ),
        mesh=mesh,
        scratch_types=[
            pltpu.VMEM((IBLK, CHUNK), jnp.int32),
            pltpu.VMEM((IBLK, CHUNK), jnp.int32),
            [pltpu.VMEM((CHUNK, D), jnp.float32) for _ in range(NBUF)],
            pltpu.VMEM((CHUNK, CW), jnp.float32),
            pltpu.VMEM_SHARED((N, D), jnp.float32),
            pltpu.VMEM_SHARED((N, CW), jnp.float32),
            [pltpu.SemaphoreType.DMA for _ in range(NBUF)],
            [pltpu.SemaphoreType.DMA for _ in range(NBUF)],
            pltpu.SemaphoreType.DMA,
        ],
        compiler_params=pltpu.CompilerParams(use_tc_tiling_on_sc=False),
    )
    def k(x1_hbm, x2_hbm, ei1_hbm, ei2_hbm, agg_hbm, cnt_hbm,
          src_v, dst_v, rows, ones_v, agg_sh, cnt_sh, gsem, ssem, osem):
        c = lax.axis_index("c")
        s = lax.axis_index("s")

        z16 = jnp.zeros((LANES,), jnp.float32)
        for r in range(CHUNK):
            for q in range(D // LANES):
                rows[0][r, pl.ds(q * LANES, LANES)] = z16
            ones_v[r, pl.ds(0, LANES)] = z16
        base = s * rows_per_tile
        for i in range(rows_per_tile // CHUNK):
            pltpu.sync_copy(rows[0], agg_sh.at[pl.ds(base + i * CHUNK, CHUNK)])
            pltpu.sync_copy(ones_v, cnt_sh.at[pl.ds(base + i * CHUNK, CHUNK)])
        rem = rows_per_tile % CHUNK
        if rem:
            pltpu.sync_copy(
                rows[0].at[pl.ds(0, rem)],
                agg_sh.at[pl.ds(base + rows_per_tile - rem, rem)])
            pltpu.sync_copy(
                ones_v.at[pl.ds(0, rem)],
                cnt_sh.at[pl.ds(base + rows_per_tile - rem, rem)])
        o16 = jnp.ones((LANES,), jnp.float32)
        for r in range(CHUNK):
            ones_v[r, pl.ds(0, LANES)] = o16
        plsc.subcore_barrier()

        def make_outer(tab, ei):
            def outer(t, _):
                pltpu.sync_copy(ei.at[0, s].at[pl.ds(t * IBLK, IBLK)], src_v)
                pltpu.sync_copy(ei.at[1, s].at[pl.ds(t * IBLK, IBLK)], dst_v)

                def start_gather(k_):
                    b = k_ % NBUF
                    pltpu.async_copy(tab.at[src_v.at[k_]], rows[b], gsem[b])

                def wait_gather(k_):
                    b = k_ % NBUF
                    pltpu.make_async_copy(tab.at[src_v.at[k_]], rows[b],
                                          gsem[b]).wait()

                def start_scatter(k_):
                    b = k_ % NBUF
                    pltpu.async_copy(rows[b], agg_sh.at[dst_v.at[k_]],
                                     ssem[b], add=True)
                    pltpu.async_copy(ones_v, cnt_sh.at[dst_v.at[k_]],
                                     osem, add=True)

                def wait_scatter(k_):
                    b = k_ % NBUF
                    pltpu.make_async_copy(rows[b], agg_sh.at[dst_v.at[k_]],
                                          ssem[b]).wait()

                start_gather(0)
                start_gather(1)
                for k_ in range(IBLK):
                    wait_gather(k_)
                    start_scatter(k_)
                    if k_ + 2 < IBLK:
                        if k_ >= 1:
                            wait_scatter(k_ - 1)
                        start_gather(k_ + 2)
                for k_ in (IBLK - 3, IBLK - 2, IBLK - 1):
                    wait_scatter(k_)
                for k_ in range(IBLK):
                    pltpu.make_async_copy(ones_v, cnt_sh.at[dst_v.at[k_]],
                                          osem).wait()
                return ()
            return outer

        @pl.when(c == 0)
        def _():
            lax.fori_loop(0, n_iblk, make_outer(x1_hbm, ei1_hbm), ())

        @pl.when(c == 1)
        def _():
            lax.fori_loop(0, n_iblk, make_outer(x2_hbm, ei2_hbm), ())

        plsc.subcore_barrier()

        pltpu.sync_copy(
            agg_sh.at[pl.ds(s * rows_per_tile, rows_per_tile)],
            agg_hbm.at[pl.ds(c * N + s * rows_per_tile, rows_per_tile)],
        )
        pltpu.sync_copy(
            cnt_sh.at[pl.ds(s * rows_per_tile, rows_per_tile)],
            cnt_hbm.at[pl.ds(c * N + s * rows_per_tile, rows_per_tile)],
        )

    return k(x1, x2, ei1, ei2)


def _tc_finish(agg, cnt, x1, x2, W_l, b_l, W_r, N):
    BLK = 1000
    D = x1.shape[1]

    def body(agg1_ref, agg2_ref, cnt1_ref, cnt2_ref, x1_ref, x2_ref,
             wl_ref, bl_ref, wr_ref, o1_ref, o2_ref):
        dn = (((1,), (1,)), ((), ()))

        def one(agg_ref, cnt_ref, x_ref, o_ref):
            inv = 1.0 / jnp.maximum(cnt_ref[...][:, 0:1], 1.0)
            mean = agg_ref[...] * inv
            o_ref[...] = (
                lax.dot_general(mean, wl_ref[...], dn,
                                preferred_element_type=jnp.float32)
                + lax.dot_general(x_ref[...], wr_ref[...], dn,
                                  preferred_element_type=jnp.float32)
                + bl_ref[...]
            )

        one(agg1_ref, cnt1_ref, x1_ref, o1_ref)
        one(agg2_ref, cnt2_ref, x2_ref, o2_ref)

    nblk = N // BLK
    return pl.pallas_call(
        body,
        grid=(nblk,),
        in_specs=[
            pl.BlockSpec((BLK, D), lambda i: (i, 0)),
            pl.BlockSpec((BLK, D), lambda i: (i + nblk, 0)),
            pl.BlockSpec((BLK, CW), lambda i: (i, 0)),
            pl.BlockSpec((BLK, CW), lambda i: (i + nblk, 0)),
            pl.BlockSpec((BLK, D), lambda i: (i, 0)),
            pl.BlockSpec((BLK, D), lambda i: (i, 0)),
            pl.BlockSpec((D, D), lambda i: (0, 0)),
            pl.BlockSpec((1, D), lambda i: (0, 0)),
            pl.BlockSpec((D, D), lambda i: (0, 0)),
        ],
        out_specs=[
            pl.BlockSpec((BLK, D), lambda i: (i, 0)),
            pl.BlockSpec((BLK, D), lambda i: (i, 0)),
        ],
        out_shape=[
            jax.ShapeDtypeStruct((N, D), jnp.float32),
            jax.ShapeDtypeStruct((N, D), jnp.float32),
        ],
    )(agg, agg, cnt, cnt, x1, x2, W_l, b_l.reshape(1, D), W_r)


def kernel(x1, edge_index1, x2, edge_index2, W_l, b_l, W_r):
    N, D = x1.shape
    E = edge_index1.shape[1]
    n_chunks = E // NS // CHUNK

    ei1 = edge_index1.reshape(2, NS, n_chunks, CHUNK)
    ei2 = edge_index2.reshape(2, NS, n_chunks, CHUNK)
    agg, cnt = _sc_aggregate(x1, x2, ei1, ei2, N, D, E)
    o1, o2 = _tc_finish(agg, cnt, x1, x2, W_l, b_l, W_r, N)
    return o1, o2


# speedup vs baseline: 14.3911x; 1.0119x over previous
import functools

import jax
import jax.numpy as jnp
from jax import lax
from jax.experimental import pallas as pl
from jax.experimental.pallas import tpu as pltpu
from jax.experimental.pallas import tpu_sc as plsc

NC = 2
NS = 16
LANES = 16
CHUNK = 80


def _sc_aggregate(x1, x2, ei1, ei2, N, D, E):
    edges_per_tile = E // NS
    n_chunks = edges_per_tile // CHUNK
    IBLK = 25
    n_iblk = n_chunks // IBLK
    rows_per_tile = N // NS
    CSTRIP = (N // NS) & ~7
    NBUF = 3

    mesh = plsc.VectorSubcoreMesh(core_axis_name="c", subcore_axis_name="s")

    @functools.partial(
        pl.kernel,
        out_type=(
            jax.ShapeDtypeStruct((2 * N, D), jnp.float32),
            jax.ShapeDtypeStruct((2 * N,), jnp.float32),
        ),
        mesh=mesh,
        scratch_types=[
            pltpu.VMEM((IBLK, CHUNK), jnp.int32),
            pltpu.VMEM((IBLK, CHUNK), jnp.int32),
            [pltpu.VMEM((CHUNK, D), jnp.float32) for _ in range(NBUF)],
            pltpu.VMEM((CHUNK,), jnp.float32),
            pltpu.VMEM((CSTRIP,), jnp.float32),
            pltpu.VMEM_SHARED((N, D), jnp.float32),
            pltpu.VMEM_SHARED((N,), jnp.float32),
            [pltpu.SemaphoreType.DMA for _ in range(NBUF)],
            [pltpu.SemaphoreType.DMA for _ in range(NBUF)],
            pltpu.SemaphoreType.DMA,
        ],
        compiler_params=pltpu.CompilerParams(use_tc_tiling_on_sc=False),
    )
    def k(x1_hbm, x2_hbm, ei1_hbm, ei2_hbm, agg_hbm, cnt_hbm,
          src_v, dst_v, rows, ones_v, cz_v, agg_sh, cnt_sh, gsem, ssem, osem):
        c = lax.axis_index("c")
        s = lax.axis_index("s")

        z16 = jnp.zeros((LANES,), jnp.float32)
        for r in range(CHUNK):
            for q in range(D // LANES):
                rows[0][r, pl.ds(q * LANES, LANES)] = z16
        for q in range(CSTRIP // LANES):
            cz_v[pl.ds(q * LANES, LANES)] = z16
        o16 = jnp.ones((LANES,), jnp.float32)
        for q in range(CHUNK // LANES):
            ones_v[pl.ds(q * LANES, LANES)] = o16

        base = s * rows_per_tile
        for i in range(rows_per_tile // CHUNK):
            pltpu.sync_copy(rows[0], agg_sh.at[pl.ds(base + i * CHUNK, CHUNK)])
        rem = rows_per_tile % CHUNK
        if rem:
            pltpu.sync_copy(
                rows[0].at[pl.ds(0, rem)],
                agg_sh.at[pl.ds(base + rows_per_tile - rem, rem)])
        pltpu.sync_copy(cz_v, cnt_sh.at[pl.ds(s * CSTRIP, CSTRIP)])

        @pl.when(s == 0)
        def _():
            pltpu.sync_copy(cz_v.at[pl.ds(0, N - NS * CSTRIP)],
                            cnt_sh.at[pl.ds(NS * CSTRIP, N - NS * CSTRIP)])
        plsc.subcore_barrier()

        def make_outer(tab, ei):
            def outer(t, _):
                pltpu.sync_copy(ei.at[0, s].at[pl.ds(t * IBLK, IBLK)], src_v)
                pltpu.sync_copy(ei.at[1, s].at[pl.ds(t * IBLK, IBLK)], dst_v)

                def start_gather(k_):
                    b = k_ % NBUF
                    pltpu.async_copy(tab.at[src_v.at[k_]], rows[b], gsem[b])

                def wait_gather(k_):
                    b = k_ % NBUF
                    pltpu.make_async_copy(tab.at[src_v.at[k_]], rows[b],
                                          gsem[b]).wait()

                def start_scatter(k_):
                    b = k_ % NBUF
                    pltpu.async_copy(rows[b], agg_sh.at[dst_v.at[k_]],
                                     ssem[b], add=True)
                    pltpu.async_copy(ones_v, cnt_sh.at[dst_v.at[k_]],
                                     osem, add=True)

                def wait_scatter(k_):
                    b = k_ % NBUF
                    pltpu.make_async_copy(rows[b], agg_sh.at[dst_v.at[k_]],
                                          ssem[b]).wait()

                start_gather(0)
                start_gather(1)
                for k_ in range(IBLK):
                    wait_gather(k_)
                    start_scatter(k_)
                    if k_ + 2 < IBLK:
                        if k_ >= 1:
                            wait_scatter(k_ - 1)
                        start_gather(k_ + 2)
                for k_ in (IBLK - 3, IBLK - 2, IBLK - 1):
                    wait_scatter(k_)
                for k_ in range(IBLK):
                    pltpu.make_async_copy(ones_v, cnt_sh.at[dst_v.at[k_]],
                                          osem).wait()
                return ()
            return outer

        @pl.when(c == 0)
        def _():
            lax.fori_loop(0, n_iblk, make_outer(x1_hbm, ei1_hbm), ())

        @pl.when(c == 1)
        def _():
            lax.fori_loop(0, n_iblk, make_outer(x2_hbm, ei2_hbm), ())

        plsc.subcore_barrier()

        pltpu.sync_copy(
            agg_sh.at[pl.ds(s * rows_per_tile, rows_per_tile)],
            agg_hbm.at[pl.ds(c * N + s * rows_per_tile, rows_per_tile)],
        )
        pltpu.sync_copy(
            cnt_sh.at[pl.ds(s * CSTRIP, CSTRIP)],
            cnt_hbm.at[pl.ds(c * N + s * CSTRIP, CSTRIP)],
        )

        @pl.when(s == 0)
        def _():
            pltpu.sync_copy(
                cnt_sh.at[pl.ds(NS * CSTRIP, N - NS * CSTRIP)],
                cnt_hbm.at[pl.ds(c * N + NS * CSTRIP, N - NS * CSTRIP)],
            )

    return k(x1, x2, ei1, ei2)


def _tc_finish(agg, cnt, x1, x2, W_l, b_l, W_r, N):
    BLK = 1000
    D = x1.shape[1]

    def body(agg1_ref, agg2_ref, cnt1_ref, cnt2_ref, x1_ref, x2_ref,
             wl_ref, bl_ref, wr_ref, o1_ref, o2_ref):
        dn = (((1,), (1,)), ((), ()))

        def one(agg_ref, cnt_ref, x_ref, o_ref):
            inv = 1.0 / jnp.maximum(cnt_ref[...], 1.0)
            mean = agg_ref[...] * inv
            o_ref[...] = (
                lax.dot_general(mean, wl_ref[...], dn,
                                preferred_element_type=jnp.float32)
                + lax.dot_general(x_ref[...], wr_ref[...], dn,
                                  preferred_element_type=jnp.float32)
                + bl_ref[...]
            )

        one(agg1_ref, cnt1_ref, x1_ref, o1_ref)
        one(agg2_ref, cnt2_ref, x2_ref, o2_ref)

    nblk = N // BLK
    return pl.pallas_call(
        body,
        grid=(nblk,),
        in_specs=[
            pl.BlockSpec((BLK, D), lambda i: (i, 0)),
            pl.BlockSpec((BLK, D), lambda i: (i + nblk, 0)),
            pl.BlockSpec((BLK, 1), lambda i: (i, 0)),
            pl.BlockSpec((BLK, 1), lambda i: (i + nblk, 0)),
            pl.BlockSpec((BLK, D), lambda i: (i, 0)),
            pl.BlockSpec((BLK, D), lambda i: (i, 0)),
            pl.BlockSpec((D, D), lambda i: (0, 0)),
            pl.BlockSpec((1, D), lambda i: (0, 0)),
            pl.BlockSpec((D, D), lambda i: (0, 0)),
        ],
        out_specs=[
            pl.BlockSpec((BLK, D), lambda i: (i, 0)),
            pl.BlockSpec((BLK, D), lambda i: (i, 0)),
        ],
        out_shape=[
            jax.ShapeDtypeStruct((N, D), jnp.float32),
            jax.ShapeDtypeStruct((N, D), jnp.float32),
        ],
    )(agg, agg, cnt, cnt, x1, x2, W_l, b_l.reshape(1, D), W_r)


def kernel(x1, edge_index1, x2, edge_index2, W_l, b_l, W_r):
    N, D = x1.shape
    E = edge_index1.shape[1]
    n_chunks = E // NS // CHUNK

    ei1 = edge_index1.reshape(2, NS, n_chunks, CHUNK)
    ei2 = edge_index2.reshape(2, NS, n_chunks, CHUNK)
    agg, cnt = _sc_aggregate(x1, x2, ei1, ei2, N, D, E)
    o1, o2 = _tc_finish(agg, cnt.reshape(2 * N, 1), x1, x2, W_l, b_l, W_r, N)
    return o1, o2


# speedup vs baseline: 15.2713x; 1.0612x over previous
import functools

import jax
import jax.numpy as jnp
from jax import lax
from jax.experimental import pallas as pl
from jax.experimental.pallas import tpu as pltpu
from jax.experimental.pallas import tpu_sc as plsc

NC = 2
NS = 16
LANES = 16
CHUNK = 80


def _sc_aggregate(x1, x2, ei1, ei2, N, D, E):
    edges_per_tile = E // NS
    n_chunks = edges_per_tile // CHUNK
    IBLK = 50
    n_iblk = n_chunks // IBLK
    rows_per_tile = N // NS
    CSTRIP = (N // NS) & ~7
    NBUF = 3

    mesh = plsc.VectorSubcoreMesh(core_axis_name="c", subcore_axis_name="s")

    @functools.partial(
        pl.kernel,
        out_type=(
            jax.ShapeDtypeStruct((2 * N, D), jnp.float32),
            jax.ShapeDtypeStruct((2 * N,), jnp.float32),
        ),
        mesh=mesh,
        scratch_types=[
            pltpu.VMEM((2, IBLK, CHUNK), jnp.int32),
            pltpu.VMEM((2, IBLK, CHUNK), jnp.int32),
            [pltpu.VMEM((CHUNK, D), jnp.float32) for _ in range(NBUF)],
            pltpu.VMEM((CHUNK,), jnp.float32),
            pltpu.VMEM((CSTRIP,), jnp.float32),
            pltpu.VMEM_SHARED((N, D), jnp.float32),
            pltpu.VMEM_SHARED((N,), jnp.float32),
            [pltpu.SemaphoreType.DMA for _ in range(NBUF)],
            [pltpu.SemaphoreType.DMA for _ in range(NBUF)],
            pltpu.SemaphoreType.DMA,
            pltpu.SemaphoreType.DMA,
        ],
        compiler_params=pltpu.CompilerParams(use_tc_tiling_on_sc=False),
    )
    def k(x1_hbm, x2_hbm, ei1_hbm, ei2_hbm, agg_hbm, cnt_hbm,
          src_v, dst_v, rows, ones_v, cz_v, agg_sh, cnt_sh, gsem, ssem, osem,
          isem):
        c = lax.axis_index("c")
        s = lax.axis_index("s")

        z16 = jnp.zeros((LANES,), jnp.float32)
        for r in range(CHUNK):
            for q in range(D // LANES):
                rows[0][r, pl.ds(q * LANES, LANES)] = z16
        for q in range(CSTRIP // LANES):
            cz_v[pl.ds(q * LANES, LANES)] = z16
        o16 = jnp.ones((LANES,), jnp.float32)
        for q in range(CHUNK // LANES):
            ones_v[pl.ds(q * LANES, LANES)] = o16

        base = s * rows_per_tile
        for i in range(rows_per_tile // CHUNK):
            pltpu.sync_copy(rows[0], agg_sh.at[pl.ds(base + i * CHUNK, CHUNK)])
        rem = rows_per_tile % CHUNK
        if rem:
            pltpu.sync_copy(
                rows[0].at[pl.ds(0, rem)],
                agg_sh.at[pl.ds(base + rows_per_tile - rem, rem)])
        pltpu.sync_copy(cz_v, cnt_sh.at[pl.ds(s * CSTRIP, CSTRIP)])

        @pl.when(s == 0)
        def _():
            pltpu.sync_copy(cz_v.at[pl.ds(0, N - NS * CSTRIP)],
                            cnt_sh.at[pl.ds(NS * CSTRIP, N - NS * CSTRIP)])
        plsc.subcore_barrier()

        def drain_ones(dv):
            for k_ in range(IBLK):
                pltpu.make_async_copy(ones_v, cnt_sh.at[dv.at[k_]],
                                      osem).wait()

        def prefetch(ei, t, parity):
            pltpu.async_copy(ei.at[0, s].at[pl.ds(t * IBLK, IBLK)],
                             src_v.at[parity], isem)
            pltpu.async_copy(ei.at[1, s].at[pl.ds(t * IBLK, IBLK)],
                             dst_v.at[parity], isem)

        def wait_prefetch(ei, t, parity):
            pltpu.make_async_copy(ei.at[0, s].at[pl.ds(t * IBLK, IBLK)],
                                  src_v.at[parity], isem).wait()
            pltpu.make_async_copy(ei.at[1, s].at[pl.ds(t * IBLK, IBLK)],
                                  dst_v.at[parity], isem).wait()

        def make_outer(tab, ei):
            def outer(t, _):
                p = t % 2
                sv = src_v.at[p]
                dv = dst_v.at[p]
                wait_prefetch(ei, t, p)

                @pl.when(t >= 1)
                def _():
                    drain_ones(dst_v.at[1 - p])

                @pl.when(t + 1 < n_iblk)
                def _():
                    prefetch(ei, t + 1, 1 - p)

                def start_gather(k_):
                    b = k_ % NBUF
                    pltpu.async_copy(tab.at[sv.at[k_]], rows[b], gsem[b])

                def wait_gather(k_):
                    b = k_ % NBUF
                    pltpu.make_async_copy(tab.at[sv.at[k_]], rows[b],
                                          gsem[b]).wait()

                def start_scatter(k_):
                    b = k_ % NBUF
                    pltpu.async_copy(rows[b], agg_sh.at[dv.at[k_]],
                                     ssem[b], add=True)
                    pltpu.async_copy(ones_v, cnt_sh.at[dv.at[k_]],
                                     osem, add=True)

                def wait_scatter(k_):
                    b = k_ % NBUF
                    pltpu.make_async_copy(rows[b], agg_sh.at[dv.at[k_]],
                                          ssem[b]).wait()

                start_gather(0)
                start_gather(1)
                for k_ in range(IBLK):
                    wait_gather(k_)
                    start_scatter(k_)
                    if k_ + 2 < IBLK:
                        if k_ >= 1:
                            wait_scatter(k_ - 1)
                        start_gather(k_ + 2)
                for k_ in (IBLK - 3, IBLK - 2, IBLK - 1):
                    wait_scatter(k_)
                return ()
            return outer

        @pl.when(c == 0)
        def _():
            prefetch(ei1_hbm, 0, 0)
            lax.fori_loop(0, n_iblk, make_outer(x1_hbm, ei1_hbm), ())
            drain_ones(dst_v.at[(n_iblk - 1) % 2])

        @pl.when(c == 1)
        def _():
            prefetch(ei2_hbm, 0, 0)
            lax.fori_loop(0, n_iblk, make_outer(x2_hbm, ei2_hbm), ())
            drain_ones(dst_v.at[(n_iblk - 1) % 2])

        plsc.subcore_barrier()

        pltpu.sync_copy(
            agg_sh.at[pl.ds(s * rows_per_tile, rows_per_tile)],
            agg_hbm.at[pl.ds(c * N + s * rows_per_tile, rows_per_tile)],
        )
        pltpu.sync_copy(
            cnt_sh.at[pl.ds(s * CSTRIP, CSTRIP)],
            cnt_hbm.at[pl.ds(c * N + s * CSTRIP, CSTRIP)],
        )

        @pl.when(s == 0)
        def _():
            pltpu.sync_copy(
                cnt_sh.at[pl.ds(NS * CSTRIP, N - NS * CSTRIP)],
                cnt_hbm.at[pl.ds(c * N + NS * CSTRIP, N - NS * CSTRIP)],
            )

    return k(x1, x2, ei1, ei2)


def _tc_finish(agg, cnt, x1, x2, W_l, b_l, W_r, N):
    BLK = 1000
    D = x1.shape[1]

    def body(agg1_ref, agg2_ref, cnt1_ref, cnt2_ref, x1_ref, x2_ref,
             wl_ref, bl_ref, wr_ref, o1_ref, o2_ref):
        dn = (((1,), (1,)), ((), ()))

        def one(agg_ref, cnt_ref, x_ref, o_ref):
            inv = 1.0 / jnp.maximum(cnt_ref[...], 1.0)
            mean = agg_ref[...] * inv
            o_ref[...] = (
                lax.dot_general(mean, wl_ref[...], dn,
                                preferred_element_type=jnp.float32)
                + lax.dot_general(x_ref[...], wr_ref[...], dn,
                                  preferred_element_type=jnp.float32)
                + bl_ref[...]
            )

        one(agg1_ref, cnt1_ref, x1_ref, o1_ref)
        one(agg2_ref, cnt2_ref, x2_ref, o2_ref)

    nblk = N // BLK
    return pl.pallas_call(
        body,
        grid=(nblk,),
        in_specs=[
            pl.BlockSpec((BLK, D), lambda i: (i, 0)),
            pl.BlockSpec((BLK, D), lambda i: (i + nblk, 0)),
            pl.BlockSpec((BLK, 1), lambda i: (i, 0)),
            pl.BlockSpec((BLK, 1), lambda i: (i + nblk, 0)),
            pl.BlockSpec((BLK, D), lambda i: (i, 0)),
            pl.BlockSpec((BLK, D), lambda i: (i, 0)),
            pl.BlockSpec((D, D), lambda i: (0, 0)),
            pl.BlockSpec((1, D), lambda i: (0, 0)),
            pl.BlockSpec((D, D), lambda i: (0, 0)),
        ],
        out_specs=[
            pl.BlockSpec((BLK, D), lambda i: (i, 0)),
            pl.BlockSpec((BLK, D), lambda i: (i, 0)),
        ],
        out_shape=[
            jax.ShapeDtypeStruct((N, D), jnp.float32),
            jax.ShapeDtypeStruct((N, D), jnp.float32),
        ],
    )(agg, agg, cnt, cnt, x1, x2, W_l, b_l.reshape(1, D), W_r)


def kernel(x1, edge_index1, x2, edge_index2, W_l, b_l, W_r):
    N, D = x1.shape
    E = edge_index1.shape[1]
    n_chunks = E // NS // CHUNK

    ei1 = edge_index1.reshape(2, NS, n_chunks, CHUNK)
    ei2 = edge_index2.reshape(2, NS, n_chunks, CHUNK)
    agg, cnt = _sc_aggregate(x1, x2, ei1, ei2, N, D, E)
    o1, o2 = _tc_finish(agg, cnt.reshape(2 * N, 1), x1, x2, W_l, b_l, W_r, N)
    return o1, o2


# speedup vs baseline: 15.4104x; 1.0091x over previous
import functools

import jax
import jax.numpy as jnp
from jax import lax
from jax.experimental import pallas as pl
from jax.experimental.pallas import tpu as pltpu
from jax.experimental.pallas import tpu_sc as plsc

NC = 2
NS = 16
LANES = 16
CHUNK = 80


def _sc_aggregate(x1, x2, ei1, ei2, N, D, E):
    edges_per_tile = E // NS
    n_chunks = edges_per_tile // CHUNK
    IBLK = 50
    n_iblk = n_chunks // IBLK
    rows_per_tile = N // NS
    CSTRIP = (N // NS) & ~7
    NBUF = 3

    mesh = plsc.VectorSubcoreMesh(core_axis_name="c", subcore_axis_name="s")

    @functools.partial(
        pl.kernel,
        out_type=(
            jax.ShapeDtypeStruct((2 * N, D), jnp.float32),
            jax.ShapeDtypeStruct((2 * N,), jnp.float32),
        ),
        mesh=mesh,
        scratch_types=[
            pltpu.VMEM((2, IBLK, CHUNK), jnp.int32),
            pltpu.VMEM((2, IBLK, CHUNK), jnp.int32),
            [pltpu.VMEM((CHUNK, D), jnp.float32) for _ in range(NBUF)],
            pltpu.VMEM((CHUNK,), jnp.float32),
            pltpu.VMEM((CSTRIP,), jnp.float32),
            pltpu.VMEM_SHARED((N, D), jnp.float32),
            pltpu.VMEM_SHARED((N,), jnp.float32),
            [pltpu.SemaphoreType.DMA for _ in range(NBUF)],
            [pltpu.SemaphoreType.DMA for _ in range(NBUF)],
            pltpu.SemaphoreType.DMA,
            pltpu.SemaphoreType.DMA,
            pltpu.SemaphoreType.DMA,
        ],
        compiler_params=pltpu.CompilerParams(use_tc_tiling_on_sc=False),
    )
    def k(x1_hbm, x2_hbm, ei1_hbm, ei2_hbm, agg_hbm, cnt_hbm,
          src_v, dst_v, rows, ones_v, cz_v, agg_sh, cnt_sh, gsem, ssem, osem,
          isem, zsem):
        c = lax.axis_index("c")
        s = lax.axis_index("s")

        def prefetch(ei, t, parity):
            pltpu.async_copy(ei.at[0, s].at[pl.ds(t * IBLK, IBLK)],
                             src_v.at[parity], isem)
            pltpu.async_copy(ei.at[1, s].at[pl.ds(t * IBLK, IBLK)],
                             dst_v.at[parity], isem)

        def wait_prefetch(ei, t, parity):
            pltpu.make_async_copy(ei.at[0, s].at[pl.ds(t * IBLK, IBLK)],
                                  src_v.at[parity], isem).wait()
            pltpu.make_async_copy(ei.at[1, s].at[pl.ds(t * IBLK, IBLK)],
                                  dst_v.at[parity], isem).wait()

        @pl.when(c == 0)
        def _():
            prefetch(ei1_hbm, 0, 0)

        @pl.when(c == 1)
        def _():
            prefetch(ei2_hbm, 0, 0)

        z16 = jnp.zeros((LANES,), jnp.float32)
        for r in range(CHUNK):
            for q in range(D // LANES):
                rows[0][r, pl.ds(q * LANES, LANES)] = z16
        for q in range(CSTRIP // LANES):
            cz_v[pl.ds(q * LANES, LANES)] = z16
        o16 = jnp.ones((LANES,), jnp.float32)
        for q in range(CHUNK // LANES):
            ones_v[pl.ds(q * LANES, LANES)] = o16

        base = s * rows_per_tile
        zcopies = []
        for i in range(rows_per_tile // CHUNK):
            zcopies.append((rows[0], agg_sh.at[pl.ds(base + i * CHUNK, CHUNK)]))
        rem = rows_per_tile % CHUNK
        if rem:
            zcopies.append((rows[0].at[pl.ds(0, rem)],
                            agg_sh.at[pl.ds(base + rows_per_tile - rem, rem)]))
        zcopies.append((cz_v, cnt_sh.at[pl.ds(s * CSTRIP, CSTRIP)]))
        for src, dst in zcopies:
            pltpu.async_copy(src, dst, zsem)

        @pl.when(s == 0)
        def _():
            pltpu.async_copy(cz_v.at[pl.ds(0, N - NS * CSTRIP)],
                             cnt_sh.at[pl.ds(NS * CSTRIP, N - NS * CSTRIP)],
                             zsem).wait()
        for src, dst in zcopies:
            pltpu.make_async_copy(src, dst, zsem).wait()
        plsc.subcore_barrier()

        def drain_ones(dv):
            for k_ in range(IBLK):
                pltpu.make_async_copy(ones_v, cnt_sh.at[dv.at[k_]],
                                      osem).wait()

        def make_outer(tab, ei):
            def outer(t, _):
                p = t % 2
                sv = src_v.at[p]
                dv = dst_v.at[p]
                wait_prefetch(ei, t, p)

                @pl.when(t >= 1)
                def _():
                    drain_ones(dst_v.at[1 - p])

                @pl.when(t + 1 < n_iblk)
                def _():
                    prefetch(ei, t + 1, 1 - p)

                def start_gather(k_):
                    b = k_ % NBUF
                    pltpu.async_copy(tab.at[sv.at[k_]], rows[b], gsem[b])

                def wait_gather(k_):
                    b = k_ % NBUF
                    pltpu.make_async_copy(tab.at[sv.at[k_]], rows[b],
                                          gsem[b]).wait()

                def start_scatter(k_):
                    b = k_ % NBUF
                    pltpu.async_copy(rows[b], agg_sh.at[dv.at[k_]],
                                     ssem[b], add=True)
                    pltpu.async_copy(ones_v, cnt_sh.at[dv.at[k_]],
                                     osem, add=True)

                def wait_scatter(k_):
                    b = k_ % NBUF
                    pltpu.make_async_copy(rows[b], agg_sh.at[dv.at[k_]],
                                          ssem[b]).wait()

                start_gather(0)
                start_gather(1)
                for k_ in range(IBLK):
                    wait_gather(k_)
                    start_scatter(k_)
                    if k_ + 2 < IBLK:
                        if k_ >= 1:
                            wait_scatter(k_ - 1)
                        start_gather(k_ + 2)
                for k_ in (IBLK - 3, IBLK - 2, IBLK - 1):
                    wait_scatter(k_)
                return ()
            return outer

        @pl.when(c == 0)
        def _():
            lax.fori_loop(0, n_iblk, make_outer(x1_hbm, ei1_hbm), ())
            drain_ones(dst_v.at[(n_iblk - 1) % 2])

        @pl.when(c == 1)
        def _():
            lax.fori_loop(0, n_iblk, make_outer(x2_hbm, ei2_hbm), ())
            drain_ones(dst_v.at[(n_iblk - 1) % 2])

        plsc.subcore_barrier()

        wcopies = [
            (agg_sh.at[pl.ds(s * rows_per_tile, rows_per_tile)],
             agg_hbm.at[pl.ds(c * N + s * rows_per_tile, rows_per_tile)]),
            (cnt_sh.at[pl.ds(s * CSTRIP, CSTRIP)],
             cnt_hbm.at[pl.ds(c * N + s * CSTRIP, CSTRIP)]),
        ]
        for src, dst in wcopies:
            pltpu.async_copy(src, dst, zsem)

        @pl.when(s == 0)
        def _():
            pltpu.async_copy(
                cnt_sh.at[pl.ds(NS * CSTRIP, N - NS * CSTRIP)],
                cnt_hbm.at[pl.ds(c * N + NS * CSTRIP, N - NS * CSTRIP)],
                zsem).wait()
        for src, dst in wcopies:
            pltpu.make_async_copy(src, dst, zsem).wait()

    return k(x1, x2, ei1, ei2)


def _tc_finish(agg, cnt, x1, x2, W_l, b_l, W_r, N):
    BLK = 1000
    D = x1.shape[1]

    def body(agg1_ref, agg2_ref, cnt1_ref, cnt2_ref, x1_ref, x2_ref,
             wl_ref, bl_ref, wr_ref, o1_ref, o2_ref):
        dn = (((1,), (1,)), ((), ()))

        def one(agg_ref, cnt_ref, x_ref, o_ref):
            inv = 1.0 / jnp.maximum(cnt_ref[...], 1.0)
            mean = agg_ref[...] * inv
            o_ref[...] = (
                lax.dot_general(mean, wl_ref[...], dn,
                                preferred_element_type=jnp.float32)
                + lax.dot_general(x_ref[...], wr_ref[...], dn,
                                  preferred_element_type=jnp.float32)
                + bl_ref[...]
            )

        one(agg1_ref, cnt1_ref, x1_ref, o1_ref)
        one(agg2_ref, cnt2_ref, x2_ref, o2_ref)

    nblk = N // BLK
    return pl.pallas_call(
        body,
        grid=(nblk,),
        in_specs=[
            pl.BlockSpec((BLK, D), lambda i: (i, 0)),
            pl.BlockSpec((BLK, D), lambda i: (i + nblk, 0)),
            pl.BlockSpec((BLK, 1), lambda i: (i, 0)),
            pl.BlockSpec((BLK, 1), lambda i: (i + nblk, 0)),
            pl.BlockSpec((BLK, D), lambda i: (i, 0)),
            pl.BlockSpec((BLK, D), lambda i: (i, 0)),
            pl.BlockSpec((D, D), lambda i: (0, 0)),
            pl.BlockSpec((1, D), lambda i: (0, 0)),
            pl.BlockSpec((D, D), lambda i: (0, 0)),
        ],
        out_specs=[
            pl.BlockSpec((BLK, D), lambda i: (i, 0)),
            pl.BlockSpec((BLK, D), lambda i: (i, 0)),
        ],
        out_shape=[
            jax.ShapeDtypeStruct((N, D), jnp.float32),
            jax.ShapeDtypeStruct((N, D), jnp.float32),
        ],
    )(agg, agg, cnt, cnt, x1, x2, W_l, b_l.reshape(1, D), W_r)


def kernel(x1, edge_index1, x2, edge_index2, W_l, b_l, W_r):
    N, D = x1.shape
    E = edge_index1.shape[1]
    n_chunks = E // NS // CHUNK

    ei1 = edge_index1.reshape(2, NS, n_chunks, CHUNK)
    ei2 = edge_index2.reshape(2, NS, n_chunks, CHUNK)
    agg, cnt = _sc_aggregate(x1, x2, ei1, ei2, N, D, E)
    o1, o2 = _tc_finish(agg, cnt.reshape(2 * N, 1), x1, x2, W_l, b_l, W_r, N)
    return o1, o2


# speedup vs baseline: 15.4141x; 1.0002x over previous
import functools

import jax
import jax.numpy as jnp
from jax import lax
from jax.experimental import pallas as pl
from jax.experimental.pallas import tpu as pltpu
from jax.experimental.pallas import tpu_sc as plsc

NS = 16
LANES = 16
CHUNK = 80


def _sc_aggregate(x1, x2, ei1, ei2, N, D, E):
    edges_per_tile = E // NS
    n_chunks = edges_per_tile // CHUNK
    IBLK = 50
    n_iblk = n_chunks // IBLK
    rows_per_tile = N // NS
    CSTRIP = (N // NS) & ~7
    NBUF = 3

    mesh = plsc.VectorSubcoreMesh(core_axis_name="c", subcore_axis_name="s")

    @functools.partial(
        pl.kernel,
        out_type=(
            jax.ShapeDtypeStruct((2 * N, D), jnp.float32),
            jax.ShapeDtypeStruct((2 * N,), jnp.float32),
        ),
        mesh=mesh,
        scratch_types=[
            pltpu.VMEM((2, IBLK, CHUNK), jnp.int32),
            pltpu.VMEM((2, IBLK, CHUNK), jnp.int32),
            [pltpu.VMEM((CHUNK, D), jnp.float32) for _ in range(NBUF)],
            pltpu.VMEM((CHUNK,), jnp.float32),
            pltpu.VMEM((CSTRIP,), jnp.float32),
            pltpu.VMEM_SHARED((N, D), jnp.float32),
            pltpu.VMEM_SHARED((N,), jnp.float32),
            [pltpu.SemaphoreType.DMA for _ in range(NBUF)],
            [pltpu.SemaphoreType.DMA for _ in range(NBUF)],
            pltpu.SemaphoreType.DMA,
            pltpu.SemaphoreType.DMA,
            pltpu.SemaphoreType.DMA,
        ],
        compiler_params=pltpu.CompilerParams(use_tc_tiling_on_sc=False),
    )
    def k(x1_hbm, x2_hbm, ei1_hbm, ei2_hbm, agg_hbm, cnt_hbm,
          src_v, dst_v, rows, ones_v, cz_v, agg_sh, cnt_sh, gsem, ssem, osem,
          isem, zsem):
        c = lax.axis_index("c")
        s = lax.axis_index("s")

        def prefetch(ei, t, parity):
            pltpu.async_copy(ei.at[0, s].at[pl.ds(t * IBLK, IBLK)],
                             src_v.at[parity], isem)
            pltpu.async_copy(ei.at[1, s].at[pl.ds(t * IBLK, IBLK)],
                             dst_v.at[parity], isem)

        def wait_prefetch(ei, t, parity):
            pltpu.make_async_copy(ei.at[0, s].at[pl.ds(t * IBLK, IBLK)],
                                  src_v.at[parity], isem).wait()
            pltpu.make_async_copy(ei.at[1, s].at[pl.ds(t * IBLK, IBLK)],
                                  dst_v.at[parity], isem).wait()

        @pl.when(c == 0)
        def _():
            prefetch(ei1_hbm, 0, 0)

        @pl.when(c == 1)
        def _():
            prefetch(ei2_hbm, 0, 0)

        z16 = jnp.zeros((LANES,), jnp.float32)
        for r in range(CHUNK):
            for q in range(D // LANES):
                rows[0][r, pl.ds(q * LANES, LANES)] = z16
        for q in range(CSTRIP // LANES):
            cz_v[pl.ds(q * LANES, LANES)] = z16
        o16 = jnp.ones((LANES,), jnp.float32)
        for q in range(CHUNK // LANES):
            ones_v[pl.ds(q * LANES, LANES)] = o16

        base = s * rows_per_tile
        zcopies = []
        for i in range(rows_per_tile // CHUNK):
            zcopies.append((rows[0], agg_sh.at[pl.ds(base + i * CHUNK, CHUNK)]))
        rem = rows_per_tile % CHUNK
        if rem:
            zcopies.append((rows[0].at[pl.ds(0, rem)],
                            agg_sh.at[pl.ds(base + rows_per_tile - rem, rem)]))
        zcopies.append((cz_v, cnt_sh.at[pl.ds(s * CSTRIP, CSTRIP)]))
        for src, dst in zcopies:
            pltpu.async_copy(src, dst, zsem)

        @pl.when(s == 0)
        def _():
            pltpu.async_copy(cz_v.at[pl.ds(0, N - NS * CSTRIP)],
                             cnt_sh.at[pl.ds(NS * CSTRIP, N - NS * CSTRIP)],
                             zsem).wait()
        for src, dst in zcopies:
            pltpu.make_async_copy(src, dst, zsem).wait()
        plsc.subcore_barrier()

        def drain_ones(dv):
            for k_ in range(IBLK):
                pltpu.make_async_copy(ones_v, cnt_sh.at[dv.at[k_]],
                                      osem).wait()

        def make_outer(tab, ei):
            def outer(t, _):
                p = t % 2
                sv = src_v.at[p]
                dv = dst_v.at[p]
                wait_prefetch(ei, t, p)

                @pl.when(t >= 1)
                def _():
                    drain_ones(dst_v.at[1 - p])

                @pl.when(t + 1 < n_iblk)
                def _():
                    prefetch(ei, t + 1, 1 - p)

                def start_gather(k_):
                    b = k_ % NBUF
                    pltpu.async_copy(tab.at[sv.at[k_]], rows[b], gsem[b])

                def wait_gather(k_):
                    b = k_ % NBUF
                    pltpu.make_async_copy(tab.at[sv.at[k_]], rows[b],
                                          gsem[b]).wait()

                def start_scatter(k_):
                    b = k_ % NBUF
                    pltpu.async_copy(rows[b], agg_sh.at[dv.at[k_]],
                                     ssem[b], add=True)
                    pltpu.async_copy(ones_v, cnt_sh.at[dv.at[k_]],
                                     osem, add=True)

                def wait_scatter(k_):
                    b = k_ % NBUF
                    pltpu.make_async_copy(rows[b], agg_sh.at[dv.at[k_]],
                                          ssem[b]).wait()

                start_gather(0)
                start_gather(1)
                for k_ in range(IBLK):
                    wait_gather(k_)
                    start_scatter(k_)
                    if k_ + 2 < IBLK:
                        if k_ >= 1:
                            wait_scatter(k_ - 1)
                        start_gather(k_ + 2)
                for k_ in (IBLK - 3, IBLK - 2, IBLK - 1):
                    wait_scatter(k_)
                return ()
            return outer

        @pl.when(c == 0)
        def _():
            lax.fori_loop(0, n_iblk, make_outer(x1_hbm, ei1_hbm), ())
            drain_ones(dst_v.at[(n_iblk - 1) % 2])

        @pl.when(c == 1)
        def _():
            lax.fori_loop(0, n_iblk, make_outer(x2_hbm, ei2_hbm), ())
            drain_ones(dst_v.at[(n_iblk - 1) % 2])

        plsc.subcore_barrier()

        wcopies = [
            (agg_sh.at[pl.ds(s * rows_per_tile, rows_per_tile)],
             agg_hbm.at[pl.ds(c * N + s * rows_per_tile, rows_per_tile)]),
            (cnt_sh.at[pl.ds(s * CSTRIP, CSTRIP)],
             cnt_hbm.at[pl.ds(c * N + s * CSTRIP, CSTRIP)]),
        ]
        for src, dst in wcopies:
            pltpu.async_copy(src, dst, zsem)

        @pl.when(s == 0)
        def _():
            pltpu.async_copy(
                cnt_sh.at[pl.ds(NS * CSTRIP, N - NS * CSTRIP)],
                cnt_hbm.at[pl.ds(c * N + NS * CSTRIP, N - NS * CSTRIP)],
                zsem).wait()
        for src, dst in wcopies:
            pltpu.make_async_copy(src, dst, zsem).wait()

    return k(x1, x2, ei1, ei2)


def _tc_finish(agg, cnt, x1, x2, W_l, b_l, W_r, N):
    BLK = 1000
    D = x1.shape[1]

    def body(agg1_ref, agg2_ref, cnt1_ref, cnt2_ref, x1_ref, x2_ref,
             wl_ref, bl_ref, wr_ref, o1_ref, o2_ref):
        dn = (((1,), (1,)), ((), ()))

        def one(agg_ref, cnt_ref, x_ref, o_ref):
            inv = 1.0 / jnp.maximum(cnt_ref[...], 1.0)
            mean = agg_ref[...] * inv
            o_ref[...] = (
                lax.dot_general(mean, wl_ref[...], dn,
                                preferred_element_type=jnp.float32)
                + lax.dot_general(x_ref[...], wr_ref[...], dn,
                                  preferred_element_type=jnp.float32)
                + bl_ref[...]
            )

        one(agg1_ref, cnt1_ref, x1_ref, o1_ref)
        one(agg2_ref, cnt2_ref, x2_ref, o2_ref)

    nblk = N // BLK
    return pl.pallas_call(
        body,
        grid=(nblk,),
        in_specs=[
            pl.BlockSpec((BLK, D), lambda i: (i, 0)),
            pl.BlockSpec((BLK, D), lambda i: (i + nblk, 0)),
            pl.BlockSpec((BLK, 1), lambda i: (i, 0)),
            pl.BlockSpec((BLK, 1), lambda i: (i + nblk, 0)),
            pl.BlockSpec((BLK, D), lambda i: (i, 0)),
            pl.BlockSpec((BLK, D), lambda i: (i, 0)),
            pl.BlockSpec((D, D), lambda i: (0, 0)),
            pl.BlockSpec((1, D), lambda i: (0, 0)),
            pl.BlockSpec((D, D), lambda i: (0, 0)),
        ],
        out_specs=[
            pl.BlockSpec((BLK, D), lambda i: (i, 0)),
            pl.BlockSpec((BLK, D), lambda i: (i, 0)),
        ],
        out_shape=[
            jax.ShapeDtypeStruct((N, D), jnp.float32),
            jax.ShapeDtypeStruct((N, D), jnp.float32),
        ],
    )(agg, agg, cnt, cnt, x1, x2, W_l, b_l.reshape(1, D), W_r)


def kernel(x1, edge_index1, x2, edge_index2, W_l, b_l, W_r):
    N, D = x1.shape
    E = edge_index1.shape[1]
    n_chunks = E // NS // CHUNK

    ei1 = edge_index1.reshape(2, NS, n_chunks, CHUNK)
    ei2 = edge_index2.reshape(2, NS, n_chunks, CHUNK)
    agg, cnt = _sc_aggregate(x1, x2, ei1, ei2, N, D, E)
    o1, o2 = _tc_finish(agg, cnt.reshape(2 * N, 1), x1, x2, W_l, b_l, W_r, N)
    return o1, o2
